# baseline (speedup 1.0000x reference)
import functools
import jax
import jax.numpy as jnp
from jax import lax
from jax.experimental import pallas as pl
from jax.experimental.pallas import tpu as pltpu
from jax.experimental.pallas import tpu_sc as plsc

NC = 2
NS = 16
NW = NC * NS
LANES = 16


def _embed_body(nf_ref, wn_ref, bn_ref, wa_ref, ba_ref, hs_ref, p_ref):
    x = nf_ref[0]
    emb = wn_ref.shape[2]
    for h in range(wn_ref.shape[0]):
        hs = jnp.dot(x, wn_ref[h], preferred_element_type=jnp.float32)
        hs = hs + bn_ref[h][None, :]
        hs_ref[:, h * emb:(h + 1) * emb] = hs
        ps = jnp.dot(hs, wa_ref[h, :emb, :], preferred_element_type=jnp.float32)
        pd = jnp.dot(hs, wa_ref[h, emb:, :], preferred_element_type=jnp.float32)
        p_ref[:, h] = ps[:, 0] + ba_ref[h, 0]
        p_ref[:, 2 + h] = pd[:, 0]


def _embed(nf, W_node, b_node, W_att, b_att, n, heads, emb, d):
    R = 1000
    grid = n // R
    hs_cat, proj = pl.pallas_call(
        _embed_body,
        grid=(grid,),
        in_specs=[
            pl.BlockSpec((1, R, d), lambda i: (0, i, 0)),
            pl.BlockSpec((heads, d, emb), lambda i: (0, 0, 0)),
            pl.BlockSpec((heads, emb), lambda i: (0, 0)),
            pl.BlockSpec((heads, 2 * emb, 1), lambda i: (0, 0, 0)),
            pl.BlockSpec((heads, 1), lambda i: (0, 0)),
        ],
        out_specs=[
            pl.BlockSpec((R, heads * emb), lambda i: (i, 0)),
            pl.BlockSpec((R, 2 * heads), lambda i: (i, 0)),
        ],
        out_shape=[
            jax.ShapeDtypeStruct((n, heads * emb), jnp.float32),
            jax.ShapeDtypeStruct((n, 2 * heads), jnp.float32),
        ],
    )(nf, W_node, b_node, W_att, b_att)
    return hs_cat, proj


CHUNK = 64


def _make_scores_kernel(n, e_pad):
    ept = e_pad // NW
    ch = ept // CHUNK
    mesh = plsc.VectorSubcoreMesh(core_axis_name="c", subcore_axis_name="s")

    rows_per_sub = n // NS

    @functools.partial(
        pl.kernel, mesh=mesh,
        out_type=[
            jax.ShapeDtypeStruct((e_pad,), jnp.float32),
            jax.ShapeDtypeStruct((e_pad,), jnp.float32),
            jax.ShapeDtypeStruct((n,), jnp.float32),
            jax.ShapeDtypeStruct((n,), jnp.float32),
            jax.ShapeDtypeStruct((n,), jnp.float32),
            jax.ShapeDtypeStruct((n,), jnp.float32),
        ],
        scratch_types=[
            pltpu.VMEM((ch, CHUNK), jnp.int32),
            pltpu.VMEM((ch, CHUNK), jnp.int32),
            pltpu.VMEM((ept,), jnp.float32),
            pltpu.VMEM((ept,), jnp.float32),
            pltpu.VMEM((ept,), jnp.float32),
            pltpu.VMEM((CHUNK,), jnp.float32),
            pltpu.VMEM((CHUNK,), jnp.float32),
            pltpu.VMEM((CHUNK,), jnp.float32),
            pltpu.VMEM((CHUNK,), jnp.float32),
            pltpu.VMEM((CHUNK,), jnp.float32),
            pltpu.VMEM((CHUNK,), jnp.float32),
            pltpu.VMEM_SHARED((n,), jnp.float32),
            pltpu.VMEM_SHARED((n,), jnp.float32),
            pltpu.VMEM_SHARED((n,), jnp.float32),
            pltpu.VMEM_SHARED((n,), jnp.float32),
            pltpu.VMEM_SHARED((n,), jnp.float32),
            pltpu.VMEM_SHARED((n,), jnp.float32),
            pltpu.SemaphoreType.DMA,
        ],
    )
    def scores_kernel(src3d_hbm, dst3d_hbm, m_hbm,
                      a0_hbm, a1_hbm, b0_hbm, b1_hbm, zn_hbm,
                      sc0_hbm, sc1_hbm,
                      ps0c0_hbm, ps0c1_hbm, ps1c0_hbm, ps1c1_hbm,
                      src2_v, dst2_v, m_v, s0_v, s1_v,
                      ga0_v, gb0_v, ga1_v, gb1_v, x0_v, x1_v,
                      a0_sh, a1_sh, b0_sh, b1_sh, sum0_sh, sum1_sh, sem):
        cid = lax.axis_index("c")
        sid = lax.axis_index("s")
        wid = sid * NC + cid
        base = wid * ept
        pltpu.sync_copy(src3d_hbm.at[wid], src2_v)
        pltpu.sync_copy(dst3d_hbm.at[wid], dst2_v)
        pltpu.sync_copy(m_hbm.at[pl.ds(base, ept)], m_v)

        @pl.when(sid == 0)
        def _():
            pltpu.sync_copy(a0_hbm, a0_sh)

        @pl.when(sid == 1)
        def _():
            pltpu.sync_copy(a1_hbm, a1_sh)

        @pl.when(sid == 2)
        def _():
            pltpu.sync_copy(b0_hbm, b0_sh)

        @pl.when(sid == 3)
        def _():
            pltpu.sync_copy(b1_hbm, b1_sh)

        @pl.when(sid == 4)
        def _():
            pltpu.sync_copy(zn_hbm, sum0_sh)

        @pl.when(sid == 5)
        def _():
            pltpu.sync_copy(zn_hbm, sum1_sh)

        plsc.subcore_barrier()

        def step(r, carry):
            srow = src2_v.at[r]
            drow = dst2_v.at[r]
            c1 = pltpu.async_copy(a0_sh.at[srow], ga0_v, sem)
            c2 = pltpu.async_copy(b0_sh.at[drow], gb0_v, sem)
            c3 = pltpu.async_copy(a1_sh.at[srow], ga1_v, sem)
            c4 = pltpu.async_copy(b1_sh.at[drow], gb1_v, sem)
            c1.wait(); c2.wait(); c3.wait(); c4.wait()
            for k in range(CHUNK // LANES):
                sl = pl.ds(k * LANES, LANES)
                msl = pl.ds(r * CHUNK + k * LANES, LANES)
                m16 = m_v[msl]
                for (ga, gb, x_v, s_v) in ((ga0_v, gb0_v, x0_v, s0_v),
                                           (ga1_v, gb1_v, x1_v, s1_v)):
                    x = ga[sl] + gb[sl]
                    x = jnp.where(x >= 0.0, x, 0.2 * x)
                    x = jnp.maximum(jnp.minimum(x, 2.0), -2.0)
                    x = jnp.exp(x) * m16
                    x_v[sl] = x
                    s_v[msl] = x
            pltpu.sync_copy(x0_v, sum0_sh.at[srow], add=True)
            pltpu.sync_copy(x1_v, sum1_sh.at[srow], add=True)
            return carry
        lax.fori_loop(0, ch, step, 0)

        pltpu.sync_copy(s0_v, sc0_hbm.at[pl.ds(base, ept)])
        pltpu.sync_copy(s1_v, sc1_hbm.at[pl.ds(base, ept)])
        plsc.subcore_barrier()

        @pl.when(jnp.logical_and(sid == 0, cid == 0))
        def _():
            pltpu.sync_copy(sum0_sh, ps0c0_hbm)
            pltpu.sync_copy(sum1_sh, ps1c0_hbm)

        @pl.when(jnp.logical_and(sid == 0, cid == 1))
        def _():
            pltpu.sync_copy(sum0_sh, ps0c1_hbm)
            pltpu.sync_copy(sum1_sh, ps1c1_hbm)

    return scores_kernel


def _sumreduce_body(a_ref, b_ref, c_ref, d_ref, i0_ref, i1_ref):
    i0_ref[...] = 1.0 / (a_ref[...] + b_ref[...])
    i1_ref[...] = 1.0 / (c_ref[...] + d_ref[...])


def _sumreduce(ps0c0, ps0c1, ps1c0, ps1c1, n):
    return pl.pallas_call(
        _sumreduce_body,
        out_shape=[
            jax.ShapeDtypeStruct((n,), jnp.float32),
            jax.ShapeDtypeStruct((n,), jnp.float32),
        ],
    )(ps0c0, ps0c1, ps1c0, ps1c1)


def _make_agg_kernel(n, e_pad, dcat, emb):
    ept = e_pad // NW
    ch = ept // CHUNK
    rows_per_sub = n // NS
    mesh = plsc.VectorSubcoreMesh(core_axis_name="c", subcore_axis_name="s")

    @functools.partial(
        pl.kernel, mesh=mesh,
        out_type=[
            jax.ShapeDtypeStruct((n, dcat), jnp.float32),
            jax.ShapeDtypeStruct((n, dcat), jnp.float32),
        ],
        scratch_types=[
            pltpu.VMEM((ch, CHUNK), jnp.int32),
            pltpu.VMEM((ch, CHUNK), jnp.int32),
            pltpu.VMEM((ept,), jnp.float32),
            pltpu.VMEM((ept,), jnp.float32),
            pltpu.VMEM((ept,), jnp.float32),
            pltpu.VMEM((CHUNK,), jnp.float32),
            pltpu.VMEM((CHUNK,), jnp.float32),
            pltpu.VMEM((CHUNK, dcat), jnp.float32),
            pltpu.VMEM((CHUNK, dcat), jnp.float32),
            pltpu.VMEM_SHARED((n,), jnp.float32),
            pltpu.VMEM_SHARED((n,), jnp.float32),
            pltpu.VMEM_SHARED((n, dcat), jnp.float32),
            pltpu.SemaphoreType.DMA,
        ],
    )
    def agg_kernel(src3d_hbm, dst3d_hbm, keep_hbm, sc0_hbm, sc1_hbm,
                   inv0_hbm, inv1_hbm, hs_hbm, zeros_hbm,
                   out0_hbm, out1_hbm,
                   src2_v, dst2_v, keep_v, sc0_v, sc1_v,
                   gi0_v, gi1_v, rows_v, wrows_v,
                   inv0_sh, inv1_sh, acc_sh, sem):
        cid = lax.axis_index("c")
        sid = lax.axis_index("s")
        wid = sid * NC + cid
        base = wid * ept
        pltpu.sync_copy(src3d_hbm.at[wid], src2_v)
        pltpu.sync_copy(dst3d_hbm.at[wid], dst2_v)
        pltpu.sync_copy(keep_hbm.at[pl.ds(base, ept)], keep_v)
        pltpu.sync_copy(sc0_hbm.at[pl.ds(base, ept)], sc0_v)
        pltpu.sync_copy(sc1_hbm.at[pl.ds(base, ept)], sc1_v)

        @pl.when(sid == 0)
        def _():
            pltpu.sync_copy(inv0_hbm, inv0_sh)

        @pl.when(sid == 1)
        def _():
            pltpu.sync_copy(inv1_hbm, inv1_sh)

        @pl.when(sid >= 11)
        def _():
            zsl = pl.ds((sid - 11) * 2000, 2000)
            pltpu.sync_copy(zeros_hbm.at[zsl], acc_sh.at[zsl])

        plsc.subcore_barrier()

        def step(r, carry):
            srow = src2_v.at[r]
            drow = dst2_v.at[r]
            c1 = pltpu.async_copy(hs_hbm.at[drow], rows_v, sem)
            c2 = pltpu.async_copy(inv0_sh.at[srow], gi0_v, sem)
            c3 = pltpu.async_copy(inv1_sh.at[srow], gi1_v, sem)
            c2.wait(); c3.wait()
            c1.wait()
            for k in range(CHUNK // LANES):
                sl = pl.ds(k * LANES, LANES)
                esl = pl.ds(r * CHUNK + k * LANES, LANES)
                k16 = keep_v[esl]
                w0 = sc0_v[esl] * gi0_v[sl] * k16
                w1 = sc1_v[esl] * gi1_v[sl] * k16
                for j in range(LANES):
                    lane = lax.iota(jnp.int32, LANES)
                    oh = (lane == j).astype(jnp.float32)
                    w0j = jnp.sum(w0 * oh, axis=0)
                    w1j = jnp.sum(w1 * oh, axis=0)
                    row = k * LANES + j
                    for t in range(emb // LANES):
                        csl = pl.ds(t * LANES, LANES)
                        wrows_v[row, csl] = rows_v[row, csl] * w0j
                    for t in range(emb // LANES):
                        csl = pl.ds(emb + t * LANES, LANES)
                        wrows_v[row, csl] = rows_v[row, csl] * w1j
            pltpu.sync_copy(wrows_v, acc_sh.at[srow], add=True)
            return carry
        lax.fori_loop(0, ch, step, 0)

        plsc.subcore_barrier()

        @pl.when(jnp.logical_and(sid == 0, cid == 0))
        def _():
            pltpu.sync_copy(acc_sh, out0_hbm)

        @pl.when(jnp.logical_and(sid == 0, cid == 1))
        def _():
            pltpu.sync_copy(acc_sh, out1_hbm)

    return agg_kernel


def _combine_body(a_ref, b_ref, o_ref):
    o_ref[...] = a_ref[...] + b_ref[...]


def _combine(out0, out1, n, dcat):
    R = 1000
    return pl.pallas_call(
        _combine_body,
        grid=(n // R,),
        in_specs=[
            pl.BlockSpec((R, dcat), lambda i: (i, 0)),
            pl.BlockSpec((R, dcat), lambda i: (i, 0)),
        ],
        out_specs=pl.BlockSpec((R, dcat), lambda i: (i, 0)),
        out_shape=jax.ShapeDtypeStruct((n, dcat), jnp.float32),
    )(out0, out1)


def _escore_body(r0_ref, r1_ref, m_ref, s0_ref, s1_ref):
    m = m_ref[...]
    for (r_ref, s_ref) in ((r0_ref, s0_ref), (r1_ref, s1_ref)):
        x = r_ref[...]
        x = jnp.where(x >= 0.0, x, 0.2 * x)
        x = jnp.maximum(jnp.minimum(x, 2.0), -2.0)
        s_ref[...] = jnp.exp(x) * m

def _escore(raw0, raw1, smask):
    e = raw0.shape[0]
    return pl.pallas_call(
        _escore_body,
        out_shape=[jax.ShapeDtypeStruct((e,), jnp.float32),
                   jax.ShapeDtypeStruct((e,), jnp.float32)],
    )(raw0, raw1, smask)

def _wrows_body(w0_ref, w1_ref, rows_ref, o_ref, *, emb):
    o_ref[:, :emb] = rows_ref[:, :emb] * w0_ref[...][:, None]
    o_ref[:, emb:] = rows_ref[:, emb:] * w1_ref[...][:, None]

def _wrows(w0, w1, rows, emb):
    e, dcat = rows.shape
    R = NW * CHUNK
    return pl.pallas_call(
        functools.partial(_wrows_body, emb=emb),
        grid=(e // R,),
        in_specs=[
            pl.BlockSpec((R,), lambda i: (i,)),
            pl.BlockSpec((R,), lambda i: (i,)),
            pl.BlockSpec((R, dcat), lambda i: (i, 0)),
        ],
        out_specs=pl.BlockSpec((R, dcat), lambda i: (i, 0)),
        out_shape=jax.ShapeDtypeStruct((e, dcat), jnp.float32),
    )(w0, w1, rows)

def _winv_body(s0_ref, s1_ref, g0_ref, g1_ref, k_ref, w0_ref, w1_ref):
    k = k_ref[...]
    w0_ref[...] = s0_ref[...] * g0_ref[...] * k
    w1_ref[...] = s1_ref[...] * g1_ref[...] * k

def _winv(s0, s1, g0, g1, keep):
    e = s0.shape[0]
    return pl.pallas_call(
        _winv_body,
        out_shape=[jax.ShapeDtypeStruct((e,), jnp.float32),
                   jax.ShapeDtypeStruct((e,), jnp.float32)],
    )(s0, s1, g0, g1, keep)


def kernel(node_features, edges, W_node, b_node, W_att, b_att):
    n = node_features.shape[1]
    d = node_features.shape[2]
    e = edges.shape[0]
    heads = W_node.shape[0]
    emb = W_node.shape[2]
    dcat = heads * emb

    src = edges[:, 0].astype(jnp.int32)
    dst = edges[:, 1].astype(jnp.int32)

    key = src * n + dst
    order = jnp.argsort(key)
    key_s = jnp.take(key, order)
    src_s = key_s // n
    dst_s = key_s - src_s * n
    keep = jnp.concatenate([
        jnp.ones((1,), jnp.float32),
        (key_s[1:] != key_s[:-1]).astype(jnp.float32),
    ])

    quant = NW * CHUNK
    e_pad = ((e + quant - 1) // quant) * quant
    pad = e_pad - e
    src_p = jnp.concatenate([src_s, jnp.zeros((pad,), jnp.int32)])
    dst_p = jnp.concatenate([dst_s, jnp.zeros((pad,), jnp.int32)])
    smask = jnp.concatenate([jnp.ones((e,), jnp.float32),
                             jnp.zeros((pad,), jnp.float32)])
    keep_p = jnp.concatenate([keep, jnp.zeros((pad,), jnp.float32)])
    src3d = src_p.reshape(NW, e_pad // (NW * CHUNK), CHUNK)
    dst3d = dst_p.reshape(NW, e_pad // (NW * CHUNK), CHUNK)

    hs_cat, proj = _embed(node_features, W_node, b_node, W_att, b_att,
                          n, heads, emb, d)
    a0 = proj[:, 0] + 0.0
    a1 = proj[:, 1] + 0.0
    b0 = proj[:, 2] + 0.0
    b1 = proj[:, 3] + 0.0
    raw0 = jnp.take(a0, src_p) + jnp.take(b0, dst_p)
    raw1 = jnp.take(a1, src_p) + jnp.take(b1, dst_p)
    sc0, sc1 = _escore(raw0, raw1, smask)
    sums0 = jax.ops.segment_sum(sc0, src_p, num_segments=n)
    sums1 = jax.ops.segment_sum(sc1, src_p, num_segments=n)
    inv0, inv1 = _sumreduce(sums0, jnp.zeros_like(sums0),
                            sums1, jnp.zeros_like(sums1), n)
    w0, w1 = _winv(sc0, sc1, jnp.take(inv0, src_p), jnp.take(inv1, src_p),
                   keep_p)
    rows = jnp.take(hs_cat, dst_p, axis=0)
    weighted = _wrows(w0, w1, rows, emb)
    out = jax.ops.segment_sum(weighted, src_p, num_segments=n)
    return out[None, :, :]
